# Initial kernel scaffold; baseline (speedup 1.0000x reference)
import jax
import jax.numpy as jnp
from jax.experimental import pallas as pl


def kernel(sdf, all_edges):
    raise NotImplementedError("write your pallas kernel here")


# speedup vs baseline: 124.1483x; 124.1483x over previous
import functools

import jax
import jax.numpy as jnp
from jax import lax
from jax.experimental import pallas as pl
from jax.experimental.pallas import tpu as pltpu
from jax.experimental.pallas import tpu_sc as plsc

_NC = 2
_NS = 16
_NW = _NC * _NS
_L = 16

_ROW = 128
_ROWS_PER_CHUNK = 16
_C = _ROW * _ROWS_PER_CHUNK


def _softplus_neg_abs(x):
    u = jnp.exp(-jnp.abs(x))
    t = u / (u + 2.0)
    t2 = t * t
    p = 1.0 + t2 * (1.0 / 3.0 + t2 * (0.2 + t2 * (1.0 / 7.0 + t2 * (1.0 / 9.0))))
    return 2.0 * t * p


def _bce_pair(a, b):
    pa = a > 0.0
    pb = b > 0.0
    na = a < 0.0
    nb = b < 0.0
    m = jnp.where((pa != pb) | (na != nb), 1.0, 0.0)
    t0 = jnp.where(pb, 1.0, 0.0)
    t1 = jnp.where(pa, 1.0, 0.0)
    bce = (jnp.maximum(a, 0.0) - a * t0 + _softplus_neg_abs(a)
           + jnp.maximum(b, 0.0) - b * t1 + _softplus_neg_abs(b))
    return bce, m


def _sc_body(k_chunks, sdf_hbm, src_hbm, dst_hbm, out_hbm,
             idx_a, idx_b, val_a, val_b, out_v, sem_a, sem_b):
    wid = lax.axis_index("s") * _NC + lax.axis_index("c")

    def chunk(g, carry):
        acc_l, acc_c = carry
        row0 = (wid * k_chunks + g) * _ROWS_PER_CHUNK
        pltpu.sync_copy(src_hbm.at[pl.ds(row0, _ROWS_PER_CHUNK)], idx_a)
        pltpu.sync_copy(dst_hbm.at[pl.ds(row0, _ROWS_PER_CHUNK)], idx_b)
        cps = []
        for r in range(_ROWS_PER_CHUNK):
            cps.append(pltpu.async_copy(sdf_hbm.at[idx_a.at[r]], val_a.at[r], sem_a))
            cps.append(pltpu.async_copy(sdf_hbm.at[idx_b.at[r]], val_b.at[r], sem_b))
        for cp in cps:
            cp.wait()

        def row(r, c1):
            def col(j, c2):
                al, ac = c2
                a = val_a[r, pl.ds(j * _L, _L)]
                b = val_b[r, pl.ds(j * _L, _L)]
                bce, m = _bce_pair(a, b)
                return (al + bce * m, ac + m)
            return lax.fori_loop(0, _ROW // _L, col, c1)

        return lax.fori_loop(0, _ROWS_PER_CHUNK, row, (acc_l, acc_c))

    zeros = jnp.zeros((_L,), jnp.float32)
    acc_l, acc_c = lax.fori_loop(0, k_chunks, chunk, (zeros, zeros))
    out_v[0, :] = acc_l
    out_v[1, :] = acc_c
    pltpu.sync_copy(out_v, out_hbm.at[wid])


def kernel(sdf, all_edges):
    e = all_edges.shape[0]
    per_tile = _NW * _C
    k_chunks = -(-e // per_tile)
    e_pad = k_chunks * per_tile
    src = all_edges[:, 0]
    dst = all_edges[:, 1]
    if e_pad != e:
        pad = jnp.zeros((e_pad - e,), jnp.int32)
        src = jnp.concatenate([src, pad])
        dst = jnp.concatenate([dst, pad])
    src = src.reshape(e_pad // _ROW, _ROW)
    dst = dst.reshape(e_pad // _ROW, _ROW)

    mesh = plsc.VectorSubcoreMesh(core_axis_name="c", subcore_axis_name="s")
    run = pl.kernel(
        functools.partial(_sc_body, k_chunks),
        out_type=jax.ShapeDtypeStruct((_NW, 2, _L), jnp.float32),
        mesh=mesh,
        scratch_types=[
            pltpu.VMEM((_ROWS_PER_CHUNK, _ROW), jnp.int32),
            pltpu.VMEM((_ROWS_PER_CHUNK, _ROW), jnp.int32),
            pltpu.VMEM((_ROWS_PER_CHUNK, _ROW), jnp.float32),
            pltpu.VMEM((_ROWS_PER_CHUNK, _ROW), jnp.float32),
            pltpu.VMEM((2, _L), jnp.float32),
            pltpu.SemaphoreType.DMA,
            pltpu.SemaphoreType.DMA,
        ],
    )
    parts = run(sdf, src, dst)
    loss = jnp.sum(parts[:, 0, :])
    cnt = jnp.sum(parts[:, 1, :])
    return loss / jnp.maximum(cnt, 1.0)
